# baseline (speedup 1.0000x reference)
import functools
import jax, jax.numpy as jnp
from jax.experimental import pallas as pl

def _decode_kernel(x_ref, o_ref, *, img_h, img_w):
    cx = x_ref[0, 0, 0, 0]
    cy = x_ref[0, 1, 0, 0]
    bw = x_ref[0, 2, 0, 0]
    bh = x_ref[0, 3, 0, 0]
    sc = x_ref[0, 4, 0, 0]
    dw = bw * 0.5
    dh = bh * 0.5
    x1 = jnp.clip(cx - dw, 0.0, img_w)
    y1 = jnp.clip(cy - dh, 0.0, img_h)
    x2 = jnp.clip(cx + dw, 0.0, img_w)
    y2 = jnp.clip(cy + dh, 0.0, img_h)
    lane = jax.lax.broadcasted_iota(jnp.int32, (1, 8), 1)
    row = jnp.zeros((1, 8), jnp.float32)
    for i, v in ((1, x1), (2, y1), (3, x2), (4, y2), (5, sc)):
        row = jnp.where(lane == i, v, row)
    o_ref[:, :] = row[:, :7]

def kernel(x):
    _, _, h, w = x.shape
    xs = jax.lax.slice(x, (0, 0, 0, 0), (1, 8, 8, 128))
    return pl.pallas_call(
        functools.partial(_decode_kernel, img_h=float(h), img_w=float(w)),
        grid=(1,),
        in_specs=[pl.BlockSpec((1, 8, 8, 128), lambda i: (0, 0, 0, 0))],
        out_specs=pl.BlockSpec((1, 7), lambda i: (0, 0)),
        out_shape=jax.ShapeDtypeStruct((1, 7), jnp.float32),
    )(xs)
